# baseline (speedup 1.0000x reference)
import functools

import jax
import jax.numpy as jnp
from jax import lax
from jax.experimental import pallas as pl
from jax.experimental.pallas import tpu as pltpu
from jax.experimental.pallas import tpu_sc as plsc

BATCH = 16384
SN = 1000000
EN = 100000
NC, NS, L = 2, 16, 16
NW = NC * NS
BW = BATCH // NW

S_STRIPE = 62464
S_LAST_OFF = 15 * S_STRIPE
S_LAST_MAIN = 62976
S_MAIN = 999936
E_STRIPE = 6144
E_LAST_OFF = 15 * E_STRIPE
E_LAST_MAIN = 7808
E_MAIN = 99968

mesh = plsc.VectorSubcoreMesh(core_axis_name="c", subcore_axis_name="s")


@functools.partial(
    pl.kernel, mesh=mesh,
    out_type=jax.ShapeDtypeStruct((BATCH,), jnp.float32),
    scratch_types=[
        pltpu.VMEM_SHARED((SN + 64,), jnp.float32),
        pltpu.VMEM_SHARED((EN + 96,), jnp.float32),
        pltpu.VMEM_SHARED((EN + 96,), jnp.float32),
        pltpu.VMEM((BW,), jnp.int32),
        pltpu.VMEM((BW,), jnp.int32),
        pltpu.VMEM((BW,), jnp.float32),
        pltpu.VMEM((BW,), jnp.float32),
        pltpu.VMEM((BW,), jnp.float32),
        pltpu.VMEM((BW,), jnp.float32),
        pltpu.SemaphoreType.DMA,
        pltpu.SemaphoreType.DMA,
    ],
)
def _k(stu_id_hbm, exer_id_hbm, sT_hbm, kT_hbm, dT_hbm, tails_hbm, out_hbm,
       sh_s, sh_k, sh_d, sidx_v, eidx_v, s_v, k_v, d_v, o_v, sem, isem):
    sid = lax.axis_index("s")
    wid = sid * NC + lax.axis_index("c")
    base = wid * BW
    ci_e = pltpu.async_copy(exer_id_hbm.at[pl.ds(base, BW)], eidx_v, isem)
    ci_s = pltpu.async_copy(stu_id_hbm.at[pl.ds(base, BW)], sidx_v, isem)

    def body(i, carry):
        sl = pl.ds(i * L, L)
        es = jnp.exp(-s_v[sl])
        ek = jnp.exp(-k_v[sl])
        ed = jnp.exp(-d_v[sl])
        t = (10.0 * (ek - es)) / ((1.0 + es) * ((1.0 + ek) * (1.0 + ed)))
        o_v[sl] = 1.0 / (1.0 + jnp.exp(-t))
        return carry

    def stage(src, dst, off, n):
        off = pl.multiple_of(off, 128)
        return pltpu.async_copy(
            src.at[0, pl.ds(off, n)], dst.at[pl.ds(off, n)], sem)

    @pl.when(sid < NS - 1)
    def _():
        c_k = stage(kT_hbm, sh_k, sid * E_STRIPE, E_STRIPE)
        c_d = stage(dT_hbm, sh_d, sid * E_STRIPE, E_STRIPE)
        c_s = stage(sT_hbm, sh_s, sid * S_STRIPE, S_STRIPE)
        c_k.wait()
        c_d.wait()
        plsc.subcore_barrier()
        ci_e.wait()
        g_k = pltpu.async_copy(sh_k.at[eidx_v], k_v, isem)
        g_d = pltpu.async_copy(sh_d.at[eidx_v], d_v, isem)
        c_s.wait()
        plsc.subcore_barrier()
        ci_s.wait()
        g_s1 = pltpu.async_copy(sh_s.at[sidx_v.at[pl.ds(0, BW // 2)]],
                                s_v.at[pl.ds(0, BW // 2)], isem)
        g_s2 = pltpu.async_copy(sh_s.at[sidx_v.at[pl.ds(BW // 2, BW // 2)]],
                                s_v.at[pl.ds(BW // 2, BW // 2)], isem)
        g_k.wait()
        g_d.wait()
        g_s1.wait()
        lax.fori_loop(0, BW // (2 * L), body, 0)
        g_s2.wait()
        lax.fori_loop(BW // (2 * L), BW // L, body, 0)

    @pl.when(sid == NS - 1)
    def _():
        c_k = stage(kT_hbm, sh_k, E_LAST_OFF, E_LAST_MAIN)
        c_d = stage(dT_hbm, sh_d, E_LAST_OFF, E_LAST_MAIN)
        c_kt = pltpu.async_copy(tails_hbm.at[0, pl.ds(128, 128)],
                                sh_k.at[pl.ds(E_MAIN, 128)], sem)
        c_dt = pltpu.async_copy(tails_hbm.at[0, pl.ds(256, 128)],
                                sh_d.at[pl.ds(E_MAIN, 128)], sem)
        c_s = stage(sT_hbm, sh_s, S_LAST_OFF, S_LAST_MAIN)
        c_st = pltpu.async_copy(tails_hbm.at[0, pl.ds(0, 128)],
                                sh_s.at[pl.ds(S_MAIN, 128)], sem)
        c_k.wait()
        c_d.wait()
        c_kt.wait()
        c_dt.wait()
        plsc.subcore_barrier()
        ci_e.wait()
        g_k = pltpu.async_copy(sh_k.at[eidx_v], k_v, isem)
        g_d = pltpu.async_copy(sh_d.at[eidx_v], d_v, isem)
        c_s.wait()
        c_st.wait()
        plsc.subcore_barrier()
        ci_s.wait()
        g_s1 = pltpu.async_copy(sh_s.at[sidx_v.at[pl.ds(0, BW // 2)]],
                                s_v.at[pl.ds(0, BW // 2)], isem)
        g_s2 = pltpu.async_copy(sh_s.at[sidx_v.at[pl.ds(BW // 2, BW // 2)]],
                                s_v.at[pl.ds(BW // 2, BW // 2)], isem)
        g_k.wait()
        g_d.wait()
        g_s1.wait()
        lax.fori_loop(0, BW // (2 * L), body, 0)
        g_s2.wait()
        lax.fori_loop(BW // (2 * L), BW // L, body, 0)

    pltpu.sync_copy(o_v, out_hbm.at[pl.ds(base, BW)])


@jax.jit
def kernel(stu_id, exer_id, student_emb, k_difficulty, e_discrimination):
    z96 = jnp.zeros((96,), jnp.float32)
    tails = jnp.concatenate([
        student_emb[S_MAIN:, 0], jnp.zeros((64,), jnp.float32),
        k_difficulty[E_MAIN:, 0], z96,
        e_discrimination[E_MAIN:, 0], z96,
    ]).reshape(1, 384)
    out = _k(
        stu_id.astype(jnp.int32),
        exer_id.astype(jnp.int32),
        student_emb.T,
        k_difficulty.T,
        e_discrimination.T,
        tails,
    )
    return out.reshape(BATCH, 1)


# speedup vs baseline: 1.0414x; 1.0414x over previous
import functools

import jax
import jax.numpy as jnp
from jax import lax
from jax.experimental import pallas as pl
from jax.experimental.pallas import tpu as pltpu
from jax.experimental.pallas import tpu_sc as plsc

BATCH = 16384
SN = 1000000
EN = 100000
NC, NS, L = 2, 16, 16
NW = NC * NS
BW = BATCH // NW

S_STRIPE = 62464
S_LAST_OFF = 15 * S_STRIPE
S_LAST_MAIN = 62976
S_MAIN = 999936
E_STRIPE = 6144
E_LAST_OFF = 15 * E_STRIPE
E_LAST_MAIN = 7808
E_MAIN = 99968

mesh = plsc.VectorSubcoreMesh(core_axis_name="c", subcore_axis_name="s")


@functools.partial(
    pl.kernel, mesh=mesh,
    out_type=jax.ShapeDtypeStruct((BATCH,), jnp.float32),
    scratch_types=[
        pltpu.VMEM_SHARED((SN + 64,), jnp.float32),
        pltpu.VMEM_SHARED((EN + 96,), jnp.float32),
        pltpu.VMEM_SHARED((EN + 96,), jnp.float32),
        pltpu.VMEM((BW,), jnp.int32),
        pltpu.VMEM((BW,), jnp.int32),
        pltpu.VMEM((BW,), jnp.float32),
        pltpu.VMEM((BW,), jnp.float32),
        pltpu.VMEM((BW,), jnp.float32),
        pltpu.VMEM((BW,), jnp.float32),
        pltpu.SemaphoreType.DMA,
        pltpu.SemaphoreType.DMA,
    ],
)
def _k(stu_id_hbm, exer_id_hbm, sT_hbm, kT_hbm, dT_hbm, tails_hbm, out_hbm,
       sh_s, sh_k, sh_d, sidx_v, eidx_v, s_v, k_v, d_v, o_v, sem, isem):
    sid = lax.axis_index("s")
    wid = sid * NC + lax.axis_index("c")
    base = wid * BW
    ci_e = pltpu.async_copy(exer_id_hbm.at[pl.ds(base, BW)], eidx_v, isem)
    ci_s = pltpu.async_copy(stu_id_hbm.at[pl.ds(base, BW)], sidx_v, isem)

    def stage(src, dst, off, n):
        off = pl.multiple_of(off, 128)
        return pltpu.async_copy(
            src.at[0, pl.ds(off, n)], dst.at[pl.ds(off, n)], sem)

    @pl.when(sid < NS - 1)
    def _():
        c_k = stage(kT_hbm, sh_k, sid * E_STRIPE, E_STRIPE)
        c_d = stage(dT_hbm, sh_d, sid * E_STRIPE, E_STRIPE)
        c_s = stage(sT_hbm, sh_s, sid * S_STRIPE, S_STRIPE)
        c_k.wait()
        c_d.wait()
        plsc.subcore_barrier()
        ci_e.wait()
        g_k = pltpu.async_copy(sh_k.at[eidx_v], k_v, isem)
        g_d = pltpu.async_copy(sh_d.at[eidx_v], d_v, isem)
        c_s.wait()
        plsc.subcore_barrier()
        ci_s.wait()
        g_s = pltpu.async_copy(sh_s.at[sidx_v], s_v, isem)
        g_k.wait()
        g_d.wait()
        g_s.wait()

    @pl.when(sid == NS - 1)
    def _():
        c_k = stage(kT_hbm, sh_k, E_LAST_OFF, E_LAST_MAIN)
        c_d = stage(dT_hbm, sh_d, E_LAST_OFF, E_LAST_MAIN)
        c_kt = pltpu.async_copy(tails_hbm.at[0, pl.ds(128, 128)],
                                sh_k.at[pl.ds(E_MAIN, 128)], sem)
        c_dt = pltpu.async_copy(tails_hbm.at[0, pl.ds(256, 128)],
                                sh_d.at[pl.ds(E_MAIN, 128)], sem)
        c_s = stage(sT_hbm, sh_s, S_LAST_OFF, S_LAST_MAIN)
        c_st = pltpu.async_copy(tails_hbm.at[0, pl.ds(0, 128)],
                                sh_s.at[pl.ds(S_MAIN, 128)], sem)
        c_k.wait()
        c_d.wait()
        c_kt.wait()
        c_dt.wait()
        plsc.subcore_barrier()
        ci_e.wait()
        g_k = pltpu.async_copy(sh_k.at[eidx_v], k_v, isem)
        g_d = pltpu.async_copy(sh_d.at[eidx_v], d_v, isem)
        c_s.wait()
        c_st.wait()
        plsc.subcore_barrier()
        ci_s.wait()
        g_s = pltpu.async_copy(sh_s.at[sidx_v], s_v, isem)
        g_k.wait()
        g_d.wait()
        g_s.wait()

    def body(i, carry):
        sl = pl.ds(i * L, L)
        es = jnp.exp(-s_v[sl])
        ek = jnp.exp(-k_v[sl])
        ed = jnp.exp(-d_v[sl])
        t = (10.0 * (ek - es)) / ((1.0 + es) * ((1.0 + ek) * (1.0 + ed)))
        o_v[sl] = 1.0 / (1.0 + jnp.exp(-t))
        return carry

    lax.fori_loop(0, BW // L, body, 0)
    pltpu.sync_copy(o_v, out_hbm.at[pl.ds(base, BW)])


@jax.jit
def kernel(stu_id, exer_id, student_emb, k_difficulty, e_discrimination):
    z96 = jnp.zeros((96,), jnp.float32)
    tails = jnp.concatenate([
        student_emb[S_MAIN:, 0], jnp.zeros((64,), jnp.float32),
        k_difficulty[E_MAIN:, 0], z96,
        e_discrimination[E_MAIN:, 0], z96,
    ]).reshape(1, 384)
    out = _k(
        stu_id.astype(jnp.int32),
        exer_id.astype(jnp.int32),
        student_emb.T,
        k_difficulty.T,
        e_discrimination.T,
        tails,
    )
    return out.reshape(BATCH, 1)
